# baseline (speedup 1.0000x reference)
import numpy as np

import jax
import jax.numpy as jnp
from jax.experimental import pallas as pl
from jax.experimental.pallas import tpu as pltpu


_CIN1, _COUT1, _K1 = 3, 3, 5
_COUT2, _K2 = 5, 3
_H, _W = 32, 32
_OH1, _OW1 = _H - _K1 + 1, _W - _K1 + 1
_PH1, _PW1 = _OH1 // 2, _OW1 // 2
_OH2, _OW2 = _PH1 - _K2 + 1, _PW1 - _K2 + 1
_PH2, _PW2 = _OH2 // 2, _OW2 // 2
_NFEAT = _COUT2 * _PH2 * _PW2
_NHID, _NOUT = 100, 10

_RB1 = 4
_NB1 = _OH1 // _RB1
_XR1 = _RB1 + _K1 - 1
_MO1 = 2 * (_RB1 // 2) * _OW1
_M1 = _COUT1 * _MO1
_KC1 = _XR1 * _W
_PW1P = 16
_NB2 = _OH2 // 2
_MO2 = 2 * _OW2
_M2 = _COUT2 * _MO2
_KC2 = _CIN1 * (_K2 + 1) * _PW1P


def _mask1() -> np.ndarray:
    m = np.zeros((_K1 * _K1, 2, _RB1 // 2, _OW1, _XR1, _W), np.float32)
    for kh in range(_K1):
        for kw in range(_K1):
            for p in range(2):
                for j in range(_RB1 // 2):
                    for q in range(_OW1):
                        m[kh * _K1 + kw, p, j, q, 2 * j + p + kh, q + kw] = 1.0
    return m.reshape(_K1 * _K1, _MO1, _KC1)


def _mask2() -> np.ndarray:
    m = np.zeros((_CIN1 * _K2 * _K2, 2, _OW2, _CIN1, _K2 + 1, _PW1P),
                 np.float32)
    for ci in range(_CIN1):
        for kh in range(_K2):
            for kw in range(_K2):
                for p in range(2):
                    for q in range(_OW2):
                        m[(ci * _K2 + kh) * _K2 + kw, p, q,
                          ci, p + kh, q + kw] = 1.0
    return m.reshape(_CIN1 * _K2 * _K2, _MO2, _KC2)


_MASK1 = _mask1()
_MASK2 = _mask2()


def _net_kernel(w1_ref, b1_ref, w2_ref, b2_ref,
                x_ref, m1_ref, m2_ref, wf1_ref, bf1_ref, wf2_ref, bf2_ref,
                o_ref, wb1_ref, wb2_ref, p1_ref, f2_ref):
    tn = x_ref.shape[-1]

    @pl.when(pl.program_id(0) == 0)
    def _build():
        for ci in range(_CIN1):
            for co in range(_COUT1):
                acc = None
                for t in range(_K1 * _K1):
                    term = m1_ref[t] * w1_ref[(co * _CIN1 + ci) * _K1 * _K1 + t]
                    acc = term if acc is None else acc + term
                wb1_ref[ci, co * _MO1:(co + 1) * _MO1, :] = acc
        for co in range(_COUT2):
            acc = None
            for t in range(_CIN1 * _K2 * _K2):
                term = m2_ref[t] * w2_ref[co * _CIN1 * _K2 * _K2 + t]
                acc = term if acc is None else acc + term
            wb2_ref[co * _MO2:(co + 1) * _MO2, :] = acc
        p1_ref[:, :, _PW1:_PW1P, :] = jnp.zeros(
            (_CIN1, _PH1, _PW1P - _PW1, tn), jnp.float32)

    for blk in range(_NB1):
        z = None
        for ci in range(_CIN1):
            xs = x_ref[ci, _RB1 * blk:_RB1 * blk + _XR1, :, :].reshape(_KC1, tn)
            t = jnp.dot(wb1_ref[ci], xs, preferred_element_type=jnp.float32)
            z = t if z is None else z + t
        z = z.reshape(_COUT1, 2, (_RB1 // 2) * _OW1, tn)
        zh = jnp.maximum(z[:, 0], z[:, 1])
        zh = zh.reshape(_COUT1, _RB1 // 2, _PW1, 2, tn)
        zp = jnp.maximum(zh[:, :, :, 0], zh[:, :, :, 1])
        for co in range(_COUT1):
            p1_ref[co, 2 * blk:2 * blk + 2, 0:_PW1, :] = jnp.maximum(
                zp[co] + b1_ref[co], 0.0)

    for blk in range(_NB2):
        f1 = p1_ref[:, 2 * blk:2 * blk + _K2 + 1, :, :].reshape(_KC2, tn)
        z2 = jnp.dot(wb2_ref[...], f1, preferred_element_type=jnp.float32)
        z2 = z2.reshape(_COUT2, 2, _OW2, tn)
        zh2 = jnp.maximum(z2[:, 0], z2[:, 1])
        zh2 = zh2.reshape(_COUT2, _PW2, 2, tn)
        zp2 = jnp.maximum(zh2[:, :, 0], zh2[:, :, 1])
        for co in range(_COUT2):
            f2_ref[co, blk, :, :] = jnp.maximum(zp2[co] + b2_ref[co], 0.0)

    feats = f2_ref[...].reshape(_NFEAT, tn)

    h = jnp.dot(wf1_ref[...], feats, preferred_element_type=jnp.float32)
    h = jnp.maximum(h + bf1_ref[...], 0.0)
    o = jnp.dot(wf2_ref[...], h, preferred_element_type=jnp.float32)
    o_ref[...] = o + bf2_ref[...]


def kernel(x, w1, b1, w2, b2, wf1, bf1, wf2, bf2):
    n = x.shape[0]
    tile_n = n if n <= 1024 else 1024
    n_pad = ((n + tile_n - 1) // tile_n) * tile_n

    x_t = jnp.transpose(x, (1, 2, 3, 0)).astype(jnp.float32)
    if n_pad != n:
        x_t = jnp.pad(x_t, ((0, 0), (0, 0), (0, 0), (0, n_pad - n)))

    out = pl.pallas_call(
        _net_kernel,
        out_shape=jax.ShapeDtypeStruct((_NOUT, n_pad), jnp.float32),
        grid=(n_pad // tile_n,),
        in_specs=[
            pl.BlockSpec(memory_space=pltpu.MemorySpace.SMEM),
            pl.BlockSpec(memory_space=pltpu.MemorySpace.SMEM),
            pl.BlockSpec(memory_space=pltpu.MemorySpace.SMEM),
            pl.BlockSpec(memory_space=pltpu.MemorySpace.SMEM),
            pl.BlockSpec((_CIN1, _H, _W, tile_n), lambda i: (0, 0, 0, i)),
            pl.BlockSpec((_K1 * _K1, _MO1, _KC1), lambda i: (0, 0, 0)),
            pl.BlockSpec((_CIN1 * _K2 * _K2, _MO2, _KC2), lambda i: (0, 0, 0)),
            pl.BlockSpec((_NHID, _NFEAT), lambda i: (0, 0)),
            pl.BlockSpec((_NHID, 1), lambda i: (0, 0)),
            pl.BlockSpec((_NOUT, _NHID), lambda i: (0, 0)),
            pl.BlockSpec((_NOUT, 1), lambda i: (0, 0)),
        ],
        out_specs=pl.BlockSpec((_NOUT, tile_n), lambda i: (0, i)),
        scratch_shapes=[
            pltpu.VMEM((_CIN1, _M1, _KC1), jnp.float32),
            pltpu.VMEM((_M2, _KC2), jnp.float32),
            pltpu.VMEM((_CIN1, _PH1, _PW1P, tile_n), jnp.float32),
            pltpu.VMEM((_COUT2, _PH2, _PW2, tile_n), jnp.float32),
        ],
        compiler_params=pltpu.CompilerParams(
            dimension_semantics=("arbitrary",),
            vmem_limit_bytes=48 * 1024 * 1024,
        ),
    )(w1, b1, w2, b2, x_t, jnp.asarray(_MASK1), jnp.asarray(_MASK2),
      wf1, bf1, wf2, bf2)

    return out[:, :n].T


# speedup vs baseline: 1.0441x; 1.0441x over previous
import numpy as np

import jax
import jax.numpy as jnp
from jax.experimental import pallas as pl
from jax.experimental.pallas import tpu as pltpu


_CIN1, _COUT1, _K1 = 3, 3, 5
_COUT2, _K2 = 5, 3
_H, _W = 32, 32
_OH1, _OW1 = _H - _K1 + 1, _W - _K1 + 1
_PH1, _PW1 = _OH1 // 2, _OW1 // 2
_OH2, _OW2 = _PH1 - _K2 + 1, _PW1 - _K2 + 1
_PH2, _PW2 = _OH2 // 2, _OW2 // 2
_NFEAT = _COUT2 * _PH2 * _PW2
_NHID, _NOUT = 100, 10

_RB1 = 4
_NB1 = _OH1 // _RB1
_XR1 = _RB1 + _K1 - 1
_MO1 = 2 * (_RB1 // 2) * _OW1
_M1 = _COUT1 * _MO1
_KC1 = _XR1 * _W
_PW1P = 16
_NB2 = _OH2 // 2
_MO2 = 2 * _OW2
_M2 = _COUT2 * _MO2
_KC2 = _CIN1 * (_K2 + 1) * _PW1P


def _mask1() -> np.ndarray:
    m = np.zeros((_K1 * _K1, 2, _RB1 // 2, _OW1, _XR1, _W), np.float32)
    for kh in range(_K1):
        for kw in range(_K1):
            for p in range(2):
                for j in range(_RB1 // 2):
                    for q in range(_OW1):
                        m[kh * _K1 + kw, p, j, q, 2 * j + p + kh, q + kw] = 1.0
    return m.reshape(_K1 * _K1, _MO1, _KC1)


def _mask2() -> np.ndarray:
    m = np.zeros((_CIN1 * _K2 * _K2, 2, _OW2, _CIN1, _K2 + 1, _PW1P),
                 np.float32)
    for ci in range(_CIN1):
        for kh in range(_K2):
            for kw in range(_K2):
                for p in range(2):
                    for q in range(_OW2):
                        m[(ci * _K2 + kh) * _K2 + kw, p, q,
                          ci, p + kh, q + kw] = 1.0
    return m.reshape(_CIN1 * _K2 * _K2, _MO2, _KC2)


_MASK1 = _mask1()
_MASK2 = _mask2()


def _net_kernel(w1_ref, b1_ref, w2_ref, b2_ref,
                x_ref, m1_ref, m2_ref, wf1_ref, bf1_ref, wf2_ref, bf2_ref,
                o_ref, wb1_ref, wb2_ref, p1_ref, f2_ref):
    tn = x_ref.shape[-1]

    @pl.when(pl.program_id(0) == 0)
    def _build():
        for ci in range(_CIN1):
            for co in range(_COUT1):
                acc = None
                for t in range(_K1 * _K1):
                    term = m1_ref[t] * w1_ref[(co * _CIN1 + ci) * _K1 * _K1 + t]
                    acc = term if acc is None else acc + term
                wb1_ref[ci, co * _MO1:(co + 1) * _MO1, :] = acc
        for co in range(_COUT2):
            acc = None
            for t in range(_CIN1 * _K2 * _K2):
                term = m2_ref[t] * w2_ref[co * _CIN1 * _K2 * _K2 + t]
                acc = term if acc is None else acc + term
            wb2_ref[co * _MO2:(co + 1) * _MO2, :] = acc
        p1_ref[:, :, _PW1:_PW1P, :] = jnp.zeros(
            (_CIN1, _PH1, _PW1P - _PW1, tn), jnp.float32)

    for blk in range(_NB1):
        z = None
        for ci in range(_CIN1):
            xs = x_ref[ci, _RB1 * blk:_RB1 * blk + _XR1, :, :].reshape(_KC1, tn)
            t = jnp.dot(wb1_ref[ci], xs, preferred_element_type=jnp.float32)
            z = t if z is None else z + t
        z = z.reshape(_COUT1, 2, (_RB1 // 2) * _OW1, tn)
        zh = jnp.maximum(z[:, 0], z[:, 1])
        zh = zh.reshape(_COUT1, _RB1 // 2, _PW1, 2, tn)
        zp = jnp.maximum(zh[:, :, :, 0], zh[:, :, :, 1])
        for co in range(_COUT1):
            p1_ref[co, 2 * blk:2 * blk + 2, 0:_PW1, :] = jnp.maximum(
                zp[co] + b1_ref[co], 0.0)

    for blk in range(_NB2):
        f1 = p1_ref[:, 2 * blk:2 * blk + _K2 + 1, :, :].reshape(_KC2, tn)
        z2 = jnp.dot(wb2_ref[...], f1, preferred_element_type=jnp.float32)
        z2 = z2.reshape(_COUT2, 2, _OW2, tn)
        zh2 = jnp.maximum(z2[:, 0], z2[:, 1])
        zh2 = zh2.reshape(_COUT2, _PW2, 2, tn)
        zp2 = jnp.maximum(zh2[:, :, 0], zh2[:, :, 1])
        for co in range(_COUT2):
            f2_ref[co, blk, :, :] = jnp.maximum(zp2[co] + b2_ref[co], 0.0)

    feats = f2_ref[...].reshape(_NFEAT, tn)

    h = jnp.dot(wf1_ref[...], feats, preferred_element_type=jnp.float32)
    h = jnp.maximum(h + bf1_ref[...], 0.0)
    o = jnp.dot(wf2_ref[...], h, preferred_element_type=jnp.float32)
    o_ref[...] = o + bf2_ref[...]


def kernel(x, w1, b1, w2, b2, wf1, bf1, wf2, bf2):
    n = x.shape[0]
    tile_n = n if n <= 512 else 512
    n_pad = ((n + tile_n - 1) // tile_n) * tile_n

    x_t = jnp.transpose(x, (1, 2, 3, 0)).astype(jnp.float32)
    if n_pad != n:
        x_t = jnp.pad(x_t, ((0, 0), (0, 0), (0, 0), (0, n_pad - n)))

    out = pl.pallas_call(
        _net_kernel,
        out_shape=jax.ShapeDtypeStruct((_NOUT, n_pad), jnp.float32),
        grid=(n_pad // tile_n,),
        in_specs=[
            pl.BlockSpec(memory_space=pltpu.MemorySpace.SMEM),
            pl.BlockSpec(memory_space=pltpu.MemorySpace.SMEM),
            pl.BlockSpec(memory_space=pltpu.MemorySpace.SMEM),
            pl.BlockSpec(memory_space=pltpu.MemorySpace.SMEM),
            pl.BlockSpec((_CIN1, _H, _W, tile_n), lambda i: (0, 0, 0, i)),
            pl.BlockSpec((_K1 * _K1, _MO1, _KC1), lambda i: (0, 0, 0)),
            pl.BlockSpec((_CIN1 * _K2 * _K2, _MO2, _KC2), lambda i: (0, 0, 0)),
            pl.BlockSpec((_NHID, _NFEAT), lambda i: (0, 0)),
            pl.BlockSpec((_NHID, 1), lambda i: (0, 0)),
            pl.BlockSpec((_NOUT, _NHID), lambda i: (0, 0)),
            pl.BlockSpec((_NOUT, 1), lambda i: (0, 0)),
        ],
        out_specs=pl.BlockSpec((_NOUT, tile_n), lambda i: (0, i)),
        scratch_shapes=[
            pltpu.VMEM((_CIN1, _M1, _KC1), jnp.float32),
            pltpu.VMEM((_M2, _KC2), jnp.float32),
            pltpu.VMEM((_CIN1, _PH1, _PW1P, tile_n), jnp.float32),
            pltpu.VMEM((_COUT2, _PH2, _PW2, tile_n), jnp.float32),
        ],
        compiler_params=pltpu.CompilerParams(
            dimension_semantics=("arbitrary",),
            vmem_limit_bytes=48 * 1024 * 1024,
        ),
    )(w1, b1, w2, b2, x_t, jnp.asarray(_MASK1), jnp.asarray(_MASK2),
      wf1, bf1, wf2, bf2)

    return out[:, :n].T
